# Initial kernel scaffold; baseline (speedup 1.0000x reference)
import jax
import jax.numpy as jnp
from jax.experimental import pallas as pl


def kernel(inputs, embeddings):
    raise NotImplementedError("write your pallas kernel here")


# speedup vs baseline: 2.1701x; 2.1701x over previous
import functools

import jax
import jax.numpy as jnp
from jax import lax
from jax.experimental import pallas as pl
from jax.experimental.pallas import tpu as pltpu
from jax.experimental.pallas import tpu_sc as plsc

INPUT_DIM = 1000
OUTPUT_DIM = 64
ROW = OUTPUT_DIM * OUTPUT_DIM
BATCH = 4096

NUM_CORES = 2
NUM_SUBCORES = 16
NUM_WORKERS = NUM_CORES * NUM_SUBCORES
B_PER_W = BATCH // NUM_WORKERS
CHUNK = 8
NCHUNK = B_PER_W // CHUNK


def _build():
  mesh = plsc.VectorSubcoreMesh(core_axis_name="c", subcore_axis_name="s")

  @functools.partial(
      pl.kernel,
      mesh=mesh,
      out_type=jax.ShapeDtypeStruct((BATCH, ROW), jnp.float32),
      scratch_types=[
          pltpu.VMEM((B_PER_W,), jnp.int32),
          pltpu.VMEM((CHUNK, ROW), jnp.float32),
          pltpu.SemaphoreType.DMA,
      ],
  )
  def gather_kernel(idx_hbm, table_hbm, out_hbm, idx_v, buf, gsem):
    wid = lax.axis_index("s") * NUM_CORES + lax.axis_index("c")
    base = wid * B_PER_W
    pltpu.sync_copy(idx_hbm.at[pl.ds(base, B_PER_W)], idx_v)

    def body(g, carry):
      off = pl.multiple_of(g * CHUNK, CHUNK)
      pltpu.async_copy(
          table_hbm.at[idx_v.at[pl.ds(off, CHUNK)]], buf, gsem
      ).wait()
      pltpu.sync_copy(buf, out_hbm.at[pl.ds(base + off, CHUNK)])
      return carry

    lax.fori_loop(0, NCHUNK, body, 0)

  return gather_kernel


_gather = _build()


def kernel(inputs, embeddings):
  table = embeddings.reshape(INPUT_DIM, ROW)
  out = _gather(inputs, table)
  return out.reshape(BATCH, OUTPUT_DIM, OUTPUT_DIM)


# speedup vs baseline: 2.2549x; 1.0390x over previous
import functools

import jax
import jax.numpy as jnp
from jax import lax
from jax.experimental import pallas as pl
from jax.experimental.pallas import tpu as pltpu
from jax.experimental.pallas import tpu_sc as plsc

INPUT_DIM = 1000
OUTPUT_DIM = 64
ROW = OUTPUT_DIM * OUTPUT_DIM
BATCH = 4096

NUM_CORES = 2
NUM_SUBCORES = 16
NUM_WORKERS = NUM_CORES * NUM_SUBCORES
B_PER_W = BATCH // NUM_WORKERS
CHUNK = 8
NCHUNK = B_PER_W // CHUNK


def _build():
  mesh = plsc.VectorSubcoreMesh(core_axis_name="c", subcore_axis_name="s")

  @functools.partial(
      pl.kernel,
      mesh=mesh,
      out_type=jax.ShapeDtypeStruct((BATCH, ROW), jnp.float32),
      scratch_types=[
          pltpu.VMEM((B_PER_W,), jnp.int32),
          pltpu.VMEM((CHUNK, ROW), jnp.float32),
          pltpu.VMEM((CHUNK, ROW), jnp.float32),
          pltpu.SemaphoreType.DMA,
          pltpu.SemaphoreType.DMA,
          pltpu.SemaphoreType.DMA,
          pltpu.SemaphoreType.DMA,
      ],
  )
  def gather_kernel(idx_hbm, table_hbm, out_hbm, idx_v, buf0, buf1,
                    gsem0, gsem1, ssem0, ssem1):
    wid = lax.axis_index("s") * NUM_CORES + lax.axis_index("c")
    base = wid * B_PER_W
    pltpu.sync_copy(idx_hbm.at[pl.ds(base, B_PER_W)], idx_v)

    bufs = (buf0, buf1)
    gsems = (gsem0, gsem1)
    ssems = (ssem0, ssem1)

    def gather(g, b):
      return pltpu.async_copy(
          table_hbm.at[idx_v.at[pl.ds(g * CHUNK, CHUNK)]], bufs[b], gsems[b])

    def scatter(g, b):
      return pltpu.async_copy(
          bufs[b], out_hbm.at[pl.ds(base + g * CHUNK, CHUNK)], ssems[b])

    gd = [None] * NCHUNK
    sd = [None] * NCHUNK
    gd[0] = gather(0, 0)
    for g in range(NCHUNK):
      b = g % 2
      gd[g].wait()
      sd[g] = scatter(g, b)
      if g + 1 < NCHUNK:
        if g >= 1:
          sd[g - 1].wait()
        gd[g + 1] = gather(g + 1, 1 - b)
    sd[NCHUNK - 2].wait()
    sd[NCHUNK - 1].wait()

  return gather_kernel


_gather = _build()


def kernel(inputs, embeddings):
  table = embeddings.reshape(INPUT_DIM, ROW)
  out = _gather(inputs, table)
  return out.reshape(BATCH, OUTPUT_DIM, OUTPUT_DIM)
